# Initial kernel scaffold; baseline (speedup 1.0000x reference)
import jax
import jax.numpy as jnp
from jax.experimental import pallas as pl


def kernel(x, edge_index, batch, W1, b1, W2, b2):
    raise NotImplementedError("write your pallas kernel here")


# speedup vs baseline: 4.1287x; 4.1287x over previous
import functools

import jax
import jax.numpy as jnp
from jax import lax
from jax.experimental import pallas as pl
from jax.experimental.pallas import tpu as pltpu
from jax.experimental.pallas import tpu_sc as plsc

N = 10000
E = 320000
D = 128
G = 128
C = 10

NC = 2
NS = 16
NW = NC * NS

CHUNK = 64
TPC = 160
CHUNKS_PAD = NW * TPC
E_PAD = CHUNKS_PAD * CHUNK
PH = 32
NPH = TPC // PH
RPT = 640
N_ACC = NS * RPT
NBLOCK = RPT // CHUNK

_sc_mesh = plsc.VectorSubcoreMesh(
    core_axis_name="c", subcore_axis_name="s", num_cores=NC, num_subcores=NS)


@functools.partial(
    pl.kernel,
    out_type=[
        jax.ShapeDtypeStruct((NC * N_ACC, D), jnp.float32),
        jax.ShapeDtypeStruct((NC * N_ACC, 16), jnp.float32),
    ],
    mesh=_sc_mesh,
    compiler_params=pltpu.CompilerParams(use_tc_tiling_on_sc=False),
    scratch_types=[
        pltpu.VMEM((PH, CHUNK), jnp.int32),
        pltpu.VMEM((PH, CHUNK), jnp.int32),
        pltpu.VMEM((CHUNK, D), jnp.float32),
        pltpu.VMEM((CHUNK, 16), jnp.float32),
        pltpu.VMEM_SHARED((N_ACC, D), jnp.float32),
        pltpu.VMEM_SHARED((N_ACC, 16), jnp.float32),
        pltpu.SemaphoreType.DMA,
    ],
)
def _sc_agg(src_hbm, dst_hbm, x_hbm, ones_hbm, z128_hbm, z16_hbm,
            agg_out, deg_out,
            idx_s, idx_d, rows_v, ones_v, agg_sh, deg_sh, sem):
    c = lax.axis_index("c")
    s = lax.axis_index("s")
    wid = c * NS + s
    start = wid * TPC

    pltpu.sync_copy(z128_hbm, rows_v)
    pltpu.sync_copy(z16_hbm, ones_v)
    for j in range(NBLOCK):
        zsl = pl.ds(s * RPT + j * CHUNK, CHUNK)
        pltpu.sync_copy(rows_v, agg_sh.at[zsl])
        pltpu.sync_copy(ones_v, deg_sh.at[zsl])
    pltpu.sync_copy(ones_hbm, ones_v)
    plsc.subcore_barrier()

    for p in range(NPH):
        pltpu.sync_copy(src_hbm.at[pl.ds(start + p * PH, PH)], idx_s)
        pltpu.sync_copy(dst_hbm.at[pl.ds(start + p * PH, PH)], idx_d)

        def body(t, carry):
            pltpu.async_copy(x_hbm.at[idx_s.at[t]], rows_v, sem).wait()
            pltpu.sync_copy(rows_v, agg_sh.at[idx_d.at[t]], add=True)
            pltpu.sync_copy(ones_v, deg_sh.at[idx_d.at[t]], add=True)
            return carry

        lax.fori_loop(0, PH, body, 0)
    plsc.subcore_barrier()

    for j in range(NBLOCK):
        roff = s * RPT + j * CHUNK
        pltpu.sync_copy(agg_sh.at[pl.ds(roff, CHUNK)], rows_v)
        pltpu.sync_copy(rows_v, agg_out.at[pl.ds(c * N_ACC + roff, CHUNK)])
        pltpu.sync_copy(deg_sh.at[pl.ds(roff, CHUNK)], ones_v)
        pltpu.sync_copy(ones_v, deg_out.at[pl.ds(c * N_ACC + roff, CHUNK)])


RB = 400
NBLK = N // RB


def _tc_post_body(agg_ref, deg_ref, batch_ref, w1_ref, b1_ref, w2_ref, b2_ref,
                  out_ref, pooled_ref):
    i = pl.program_id(0)

    agg = agg_ref[0] + agg_ref[1]
    deg = deg_ref[0, :, 0:1] + deg_ref[1, :, 0:1]
    xm = agg / jnp.maximum(deg, 1.0)
    h = jnp.dot(xm, w1_ref[...], preferred_element_type=jnp.float32)
    h = jnp.maximum(h + b1_ref[...], 0.0)

    b = batch_ref[0]
    gids = lax.broadcasted_iota(jnp.int32, (G, 1), 0)
    oh = (b == gids).astype(jnp.float32)

    @pl.when(i == 0)
    def _():
        pooled_ref[...] = jnp.zeros_like(pooled_ref)

    pooled_ref[...] += jnp.dot(oh, h, preferred_element_type=jnp.float32)

    @pl.when(i == NBLK - 1)
    def _():
        out_ref[...] = (
            jnp.dot(pooled_ref[...], w2_ref[...],
                    preferred_element_type=jnp.float32) + b2_ref[...])


_tc_post = pl.pallas_call(
    _tc_post_body,
    grid=(NBLK,),
    in_specs=[
        pl.BlockSpec((NC, RB, D), lambda i: (0, i, 0)),
        pl.BlockSpec((NC, RB, 16), lambda i: (0, i, 0)),
        pl.BlockSpec((1, 1, RB), lambda i: (i, 0, 0)),
        pl.BlockSpec((D, D), lambda i: (0, 0)),
        pl.BlockSpec((1, D), lambda i: (0, 0)),
        pl.BlockSpec((D, C), lambda i: (0, 0)),
        pl.BlockSpec((1, C), lambda i: (0, 0)),
    ],
    out_specs=pl.BlockSpec((G, C), lambda i: (0, 0)),
    out_shape=jax.ShapeDtypeStruct((G, C), jnp.float32),
    scratch_shapes=[pltpu.VMEM((G, D), jnp.float32)],
)


@jax.jit
def kernel(x, edge_index, batch, W1, b1, W2, b2):
    npad = E_PAD - E
    src2d = jnp.concatenate(
        [edge_index[0], jnp.zeros((npad,), jnp.int32)]).reshape(CHUNKS_PAD, CHUNK)
    dst2d = jnp.concatenate(
        [edge_index[1], jnp.full((npad,), N, jnp.int32)]).reshape(CHUNKS_PAD, CHUNK)
    ones = jnp.ones((CHUNK, 16), jnp.float32)
    z128 = jnp.zeros((CHUNK, D), jnp.float32)
    z16 = jnp.zeros((CHUNK, 16), jnp.float32)
    agg2, deg2 = _sc_agg(src2d, dst2d, x, ones, z128, z16)
    agg3 = agg2.reshape(NC, N_ACC, D)
    deg3 = deg2.reshape(NC, N_ACC, 16)
    batch3d = batch.reshape(NBLK, 1, RB)
    return _tc_post(agg3, deg3, batch3d, W1, b1.reshape(1, D),
                    W2, b2.reshape(1, C))
